# baseline (speedup 1.0000x reference)
import functools

import jax
import jax.numpy as jnp
from jax import lax
from jax.experimental import pallas as pl
from jax.experimental.pallas import tpu as pltpu
from jax.experimental.pallas import tpu_sc as plsc

GAMMA = 0.1
EPSILON = 0.1

NC = 2
NS = 16
K = 128


@functools.cache
def _sc_mesh():
    return plsc.VectorSubcoreMesh(core_axis_name="core",
                                  subcore_axis_name="subcore",
                                  num_cores=NC, num_subcores=NS)


def _deg_body(npad, nch_deg, ei_hbm, ones_hbm, zeros_hbm, degp_hbm,
              ib, ones_v, zbuf, deg_sh, isems):
    stripe = npad // NS
    nd = len(ib)
    c = lax.axis_index("core")
    s = lax.axis_index("subcore")
    pltpu.sync_copy(zeros_hbm, zbuf)
    pltpu.sync_copy(zbuf, deg_sh.at[pl.ds(s * stripe, stripe)])
    pltpu.sync_copy(ones_hbm, ones_v)
    plsc.subcore_barrier()

    base = (c * NS + s) * nch_deg * K
    for t in range(nd):
        pltpu.async_copy(ei_hbm.at[:, pl.ds(base + t * K, K)], ib[t],
                         isems[t])

    @pl.loop(0, nch_deg, step=nd)
    def _(j):
        for u in range(nd):
            pltpu.make_async_copy(ei_hbm.at[:, pl.ds(base, K)], ib[u],
                                  isems[u]).wait()
            pltpu.sync_copy(ones_v, deg_sh.at[ib[u].at[1]], add=True)

            @pl.when(j + u + nd < nch_deg)
            def _():
                pltpu.async_copy(
                    ei_hbm.at[:, pl.ds(base + (j + u + nd) * K, K)], ib[u],
                    isems[u])

    plsc.subcore_barrier()
    pltpu.sync_copy(deg_sh.at[pl.ds(s * stripe, stripe)], zbuf)
    pltpu.sync_copy(zbuf, degp_hbm.at[pl.ds(c * npad + s * stripe, stripe)])


def _stage(t, u, nch, xt, ei_hbm, base, ib, gb, acc_sh, isems, gsems, ssems):
    m3, m6 = u % 3, u % 6
    n3 = (u + 1) % 3
    p6 = (u + 4) % 6
    f6 = (u + 3) % 6
    static = isinstance(t, int)

    def wait_scatter():
        pltpu.make_async_copy(gb[m3], acc_sh.at[ib[m6].at[1]],
                              ssems[m3]).wait()

    if static or u >= 3:
        wait_scatter()
    else:
        pl.when(t >= 3)(wait_scatter)

    def prefetch_idx():
        pltpu.async_copy(ei_hbm.at[:, pl.ds(base + (t + 3) * K, K)], ib[f6],
                         isems[f6])

    if static:
        if t + 3 < nch:
            prefetch_idx()
    else:
        pl.when(t + 3 < nch)(prefetch_idx)

    def scatter_prev():
        pltpu.make_async_copy(xt.at[ib[p6].at[0]], gb[n3], gsems[n3]).wait()
        pltpu.async_copy(gb[n3], acc_sh.at[ib[p6].at[1]], ssems[n3],
                         add=True)

    if static or u >= 2:
        scatter_prev()
    else:
        pl.when(t >= 2)(scatter_prev)

    pltpu.make_async_copy(ei_hbm.at[:, pl.ds(base, K)], ib[m6],
                          isems[m6]).wait()
    pltpu.async_copy(xt.at[ib[m6].at[0]], gb[m3], gsems[m3])


def _chunks(total):
    return [K] * (total // K) + ([total % K] if total % K else [])


def _gcn_body(npad, nch, xws0_hbm, xws1_hbm, ei_hbm, zeros_hbm, gcn_hbm,
              ib, gb, acc_sh, isems, gsems, ssems):
    z_main = ((npad + 8 * NS - 1) // (8 * NS)) * 8
    z_last = npad - (NS - 1) * z_main
    c = lax.axis_index("core")
    s = lax.axis_index("subcore")
    pltpu.sync_copy(zeros_hbm, gb[0])

    def zero_part(total):
        rbase = s * z_main
        off = 0
        for sz in _chunks(total):
            pltpu.sync_copy(gb[0].at[pl.ds(0, sz)],
                            acc_sh.at[pl.ds(rbase + off, sz)])
            off += sz

    @pl.when(s < NS - 1)
    def _():
        zero_part(z_main)

    @pl.when(s == NS - 1)
    def _():
        zero_part(z_last)

    plsc.subcore_barrier()
    base = s * nch * K
    main = nch - nch % 6

    def edge_loop(xt):
        for t in range(3):
            pltpu.async_copy(ei_hbm.at[:, pl.ds(base + t * K, K)], ib[t],
                             isems[t])

        @pl.loop(0, main, step=6)
        def _(j):
            for u in range(6):
                _stage(j + u, u, nch, xt, ei_hbm, base, ib, gb, acc_sh,
                       isems, gsems, ssems)

        for t in range(main, nch):
            _stage(t, t % 6, nch, xt, ei_hbm, base, ib, gb, acc_sh,
                   isems, gsems, ssems)

        pltpu.make_async_copy(gb[(nch - 3) % 3],
                              acc_sh.at[ib[(nch - 3) % 6].at[1]],
                              ssems[(nch - 3) % 3]).wait()
        for t in (nch - 2, nch - 1):
            pltpu.make_async_copy(xt.at[ib[t % 6].at[0]], gb[t % 3],
                                  gsems[t % 3]).wait()
            pltpu.sync_copy(gb[t % 3], acc_sh.at[ib[t % 6].at[1]], add=True)

    @pl.when(c == 0)
    def _():
        edge_loop(xws0_hbm)

    @pl.when(c == 1)
    def _():
        edge_loop(xws1_hbm)

    plsc.subcore_barrier()

    def copy_part(total):
        obase = s * z_main
        zs = _chunks(total)
        zoff = [sum(zs[:k]) for k in range(len(zs))]
        nz = len(zs)
        for k, (sz, off) in enumerate(zip(zs, zoff)):
            if k >= 2:
                psz, poff = zs[k - 2], zoff[k - 2]
                pltpu.make_async_copy(gb[k % 2].at[pl.ds(0, psz)],
                                      gcn_hbm.at[c, pl.ds(obase + poff, psz)],
                                      ssems[k % 2]).wait()
            pltpu.sync_copy(acc_sh.at[pl.ds(obase + off, sz)],
                            gb[k % 2].at[pl.ds(0, sz)])
            pltpu.async_copy(gb[k % 2].at[pl.ds(0, sz)],
                             gcn_hbm.at[c, pl.ds(obase + off, sz)],
                             ssems[k % 2])
        for k in (nz - 2, nz - 1):
            sz, off = zs[k], zoff[k]
            pltpu.make_async_copy(gb[k % 2].at[pl.ds(0, sz)],
                                  gcn_hbm.at[c, pl.ds(obase + off, sz)],
                                  ssems[k % 2]).wait()

    @pl.when(s < NS - 1)
    def _():
        copy_part(z_main)

    @pl.when(s == NS - 1)
    def _():
        copy_part(z_last)


def _dense_body(x_ref, wcat_ref, degp0_ref, degp1_ref, h2_ref, xws0_ref,
                xws1_ref):
    xb = x_ref[...]
    m = jnp.dot(xb.astype(jnp.bfloat16), wcat_ref[...].astype(jnp.bfloat16),
                preferred_element_type=jnp.float32)
    d = xb.shape[1]
    h2_ref[...] = m[:, d:]
    deg = degp0_ref[...] + degp1_ref[...]
    dis = jnp.where(deg > 0.0, lax.rsqrt(deg), 0.0)
    xw = m[:, :d] * dis
    half = d // 2
    xws0_ref[...] = xw[:, :half]
    xws1_ref[...] = xw[:, half:]


def _combine_body(x_ref, h2_ref, gcn_ref, degp0_ref, degp1_ref, bias_ref,
                  o_ref):
    deg = degp0_ref[...] + degp1_ref[...]
    dis = jnp.where(deg > 0.0, lax.rsqrt(deg), 0.0)
    g = jnp.concatenate([gcn_ref[0], gcn_ref[1]], axis=1)
    h = h2_ref[...] + g * dis + bias_ref[...]
    o_ref[...] = x_ref[...] + EPSILON * jnp.tanh(h)


def kernel(x, edge_index, W, W_phi, bias):
    n, d = x.shape
    e = edge_index.shape[1]
    half = d // 2
    npad = ((n + 1 + NS - 1) // NS) * NS
    stripe = npad // NS
    npad_deg = ((n + 1 + 8 * NS - 1) // (8 * NS)) * (8 * NS)
    stripe_deg = npad_deg // NS
    nch = 2 * ((e + 2 * NS * K - 1) // (2 * NS * K))
    epad = NS * K * nch
    nch_deg = epad // (NC * NS * K)

    ei = jnp.pad(edge_index.astype(jnp.int32), ((0, 0), (0, epad - e)),
                 constant_values=npad - 1)

    ones128 = jnp.ones((K,), jnp.float32)
    zeros1 = jnp.zeros((stripe_deg,), jnp.float32)
    zeros2 = jnp.zeros((K, half), jnp.float32)

    wcat = jnp.concatenate(
        [W_phi.T, (W - W.T - GAMMA * jnp.eye(d, dtype=x.dtype)).T], axis=1)

    deg_call = pl.kernel(
        functools.partial(_deg_body, npad_deg, nch_deg),
        out_type=jax.ShapeDtypeStruct((NC * npad_deg,), jnp.float32),
        mesh=_sc_mesh(),
        scratch_types=[
            [pltpu.VMEM((2, K), jnp.int32) for _ in range(4)],
            pltpu.VMEM((K,), jnp.float32),
            pltpu.VMEM((stripe_deg,), jnp.float32),
            pltpu.VMEM_SHARED((npad_deg,), jnp.float32),
            [pltpu.SemaphoreType.DMA for _ in range(4)],
        ],
    )
    degp = deg_call(ei, ones128, zeros1)
    degp0 = degp[:npad_deg].reshape(npad_deg, 1)
    degp1 = degp[npad_deg:].reshape(npad_deg, 1)

    nb = 10
    r = n // nb
    h2, xws0, xws1 = pl.pallas_call(
        _dense_body,
        grid=(nb,),
        in_specs=[
            pl.BlockSpec((r, d), lambda i: (i, 0)),
            pl.BlockSpec((d, 2 * d), lambda i: (0, 0)),
            pl.BlockSpec((r, 1), lambda i: (i, 0)),
            pl.BlockSpec((r, 1), lambda i: (i, 0)),
        ],
        out_specs=[
            pl.BlockSpec((r, d), lambda i: (i, 0)),
            pl.BlockSpec((r, half), lambda i: (i, 0)),
            pl.BlockSpec((r, half), lambda i: (i, 0)),
        ],
        out_shape=[
            jax.ShapeDtypeStruct((n, d), jnp.float32),
            jax.ShapeDtypeStruct((npad, half), jnp.float32),
            jax.ShapeDtypeStruct((npad, half), jnp.float32),
        ],
    )(x, wcat, degp0, degp1)

    gcn_call = pl.kernel(
        functools.partial(_gcn_body, npad, nch),
        out_type=jax.ShapeDtypeStruct((NC, npad, half), jnp.float32),
        mesh=_sc_mesh(),
        scratch_types=[
            [pltpu.VMEM((2, K), jnp.int32) for _ in range(6)],
            [pltpu.VMEM((K, half), jnp.float32) for _ in range(3)],
            pltpu.VMEM_SHARED((npad, half), jnp.float32),
            [pltpu.SemaphoreType.DMA for _ in range(6)],
            [pltpu.SemaphoreType.DMA for _ in range(3)],
            [pltpu.SemaphoreType.DMA for _ in range(3)],
        ],
    )
    gcn = gcn_call(xws0, xws1, ei, zeros2)

    out = pl.pallas_call(
        _combine_body,
        grid=(nb,),
        in_specs=[
            pl.BlockSpec((r, d), lambda i: (i, 0)),
            pl.BlockSpec((r, d), lambda i: (i, 0)),
            pl.BlockSpec((2, r, half), lambda i: (0, i, 0)),
            pl.BlockSpec((r, 1), lambda i: (i, 0)),
            pl.BlockSpec((r, 1), lambda i: (i, 0)),
            pl.BlockSpec((1, d), lambda i: (0, 0)),
        ],
        out_specs=pl.BlockSpec((r, d), lambda i: (i, 0)),
        out_shape=jax.ShapeDtypeStruct((n, d), jnp.float32),
    )(x, h2, gcn, degp0, degp1, bias.reshape(1, d))
    return out


# speedup vs baseline: 1.7212x; 1.7212x over previous
import functools

import jax
import jax.numpy as jnp
from jax import lax
from jax.experimental import pallas as pl
from jax.experimental.pallas import tpu as pltpu
from jax.experimental.pallas import tpu_sc as plsc

GAMMA = 0.1
EPSILON = 0.1

NC = 2
NS = 16
K = 120
NI = 6
NG = 3
ZR = 80


@functools.cache
def _sc_mesh():
    return plsc.VectorSubcoreMesh(core_axis_name="core",
                                  subcore_axis_name="subcore",
                                  num_cores=NC, num_subcores=NS)


def _deg_body(npad, nch_deg, cols_hbm, ones_hbm, zeros_hbm, degp_hbm,
              cols_v, ones_v, zbuf, deg_sh):
    stripe = npad // NS
    c = lax.axis_index("core")
    s = lax.axis_index("subcore")
    pltpu.sync_copy(zeros_hbm, zbuf)
    pltpu.sync_copy(zbuf, deg_sh.at[pl.ds(s * stripe, stripe)])
    pltpu.sync_copy(cols_hbm.at[c, s], cols_v)
    pltpu.sync_copy(ones_hbm, ones_v)
    plsc.subcore_barrier()

    @pl.loop(0, nch_deg)
    def _(j):
        pltpu.sync_copy(ones_v.at[pl.ds(0, K)], deg_sh.at[cols_v.at[j]],
                        add=True)

    plsc.subcore_barrier()
    pltpu.sync_copy(deg_sh.at[pl.ds(s * stripe, stripe)], zbuf)
    pltpu.sync_copy(zbuf, degp_hbm.at[pl.ds(c * npad + s * stripe, stripe)])


def _gcn_body(npad, nch, zrows, xws_hbm, idx_hbm, zeros_hbm, gcn_hbm,
              ib, gb, acc_sh, isems, gsems, ssems):
    stripe = npad // NS
    ni = len(ib)
    ng = len(gb)
    c = lax.axis_index("core")
    s = lax.axis_index("subcore")
    pltpu.sync_copy(zeros_hbm, gb[0].at[pl.ds(0, zrows)])

    @pl.loop(0, stripe, step=zrows)
    def _(i):
        pltpu.sync_copy(gb[0].at[pl.ds(0, zrows)],
                        acc_sh.at[pl.ds(s * stripe + i, zrows)])

    plsc.subcore_barrier()

    for t in range(ni // 2):
        pltpu.async_copy(idx_hbm.at[c, s, t], ib[t], isems[t])

    @pl.loop(0, nch, step=ni)
    def _(j):
        for u in range(ni):
            t = j + u
            tg = (u + 1) % ng
            if u >= 3:
                pltpu.make_async_copy(gb[u % ng],
                                      acc_sh.at[ib[(u + 3) % ni].at[1]],
                                      ssems[u % ng]).wait()
            else:
                @pl.when(t >= 3)
                def _():
                    pltpu.make_async_copy(gb[u % ng],
                                          acc_sh.at[ib[(u + 3) % ni].at[1]],
                                          ssems[u % ng]).wait()
            nxt = t + ni // 2
            iu = (u + ni // 2) % ni

            @pl.when(nxt < nch)
            def _():
                pltpu.async_copy(idx_hbm.at[c, s, nxt], ib[iu], isems[iu])

            if u >= 2:
                pltpu.make_async_copy(xws_hbm.at[ib[(u - 2) % ni].at[0]],
                                      gb[tg], gsems[tg]).wait()
                pltpu.async_copy(gb[tg], acc_sh.at[ib[(u - 2) % ni].at[1]],
                                 ssems[tg], add=True)
            else:
                @pl.when(t >= 2)
                def _():
                    pltpu.make_async_copy(xws_hbm.at[ib[(u - 2) % ni].at[0]],
                                          gb[tg], gsems[tg]).wait()
                    pltpu.async_copy(gb[tg],
                                     acc_sh.at[ib[(u - 2) % ni].at[1]],
                                     ssems[tg], add=True)

            pltpu.make_async_copy(idx_hbm.at[c, s, t], ib[u % ni],
                                  isems[u % ni]).wait()
            pltpu.async_copy(xws_hbm.at[ib[u % ni].at[0]], gb[u % ng],
                             gsems[u % ng])

    pltpu.make_async_copy(gb[(nch - 3) % ng],
                          acc_sh.at[ib[(nch - 3) % ni].at[1]],
                          ssems[(nch - 3) % ng]).wait()
    for t in (nch - 2, nch - 1):
        pltpu.make_async_copy(xws_hbm.at[ib[t % ni].at[0]], gb[t % ng],
                              gsems[t % ng]).wait()
        pltpu.sync_copy(gb[t % ng], acc_sh.at[ib[t % ni].at[1]], add=True)

    plsc.subcore_barrier()

    @pl.loop(0, stripe, step=2 * zrows)
    def _(i):
        pltpu.sync_copy(acc_sh.at[pl.ds(s * stripe + i, zrows)],
                        gb[0].at[pl.ds(0, zrows)])
        pltpu.sync_copy(gb[0].at[pl.ds(0, zrows)],
                        gcn_hbm.at[c, pl.ds(s * stripe + i, zrows)])
        pltpu.sync_copy(acc_sh.at[pl.ds(s * stripe + i + zrows, zrows)],
                        gb[1].at[pl.ds(0, zrows)])
        pltpu.sync_copy(gb[1].at[pl.ds(0, zrows)],
                        gcn_hbm.at[c, pl.ds(s * stripe + i + zrows, zrows)])


def _dense_body(x_ref, wcat_ref, degp_ref, h2_ref, xws_ref):
    xb = x_ref[...]
    m = jnp.dot(xb, wcat_ref[...], preferred_element_type=jnp.float32)
    d = xb.shape[1]
    h2_ref[...] = m[:, d:]
    deg = degp_ref[:, 0:1] + degp_ref[:, 1:2]
    dis = jnp.where(deg > 0.0, lax.rsqrt(deg), 0.0)
    xw = m[:, :d] * dis
    half = d // 2
    xws_ref[0] = xw[:, :half]
    xws_ref[1] = xw[:, half:]


def _combine_body(x_ref, h2_ref, gcn_ref, degp_ref, bias_ref, o_ref):
    deg = degp_ref[:, 0:1] + degp_ref[:, 1:2]
    dis = jnp.where(deg > 0.0, lax.rsqrt(deg), 0.0)
    g = jnp.concatenate([gcn_ref[0], gcn_ref[1]], axis=1)
    h = h2_ref[...] + g * dis + bias_ref[...]
    o_ref[...] = x_ref[...] + EPSILON * jnp.tanh(h)


def kernel(x, edge_index, W, W_phi, bias):
    n, d = x.shape
    e = edge_index.shape[1]
    half = d // 2
    npad = ((n + 2 * ZR * NS - 1) // (2 * ZR * NS)) * (2 * ZR * NS)
    stripe = npad // NS
    nch = NI * ((e + NI * NS * K - 1) // (NI * NS * K))
    epad = NS * K * nch
    nch_deg = epad // (NC * NS * K)

    ei = edge_index.astype(jnp.int32)
    rows = jnp.concatenate([ei[0], jnp.zeros((epad - e,), jnp.int32)])
    cols = jnp.concatenate([ei[1],
                            jnp.full((epad - e,), npad - 1, jnp.int32)])
    rows3 = rows.reshape(NS, nch, K)
    cols3 = cols.reshape(NS, nch, K)
    idx_pack = jnp.stack(
        [jnp.stack([rows3, cols3], axis=2),
         jnp.stack([rows3 + n, cols3], axis=2)], axis=0)
    cols_deg = cols.reshape(NC, NS, nch_deg, K)

    ones128 = jnp.ones((128,), jnp.float32)
    zeros1 = jnp.zeros((stripe,), jnp.float32)
    zeros2 = jnp.zeros((ZR, half), jnp.float32)

    wcat = jnp.concatenate(
        [W_phi.T, (W - W.T - GAMMA * jnp.eye(d, dtype=x.dtype)).T], axis=1)

    deg_call = pl.kernel(
        functools.partial(_deg_body, npad, nch_deg),
        out_type=jax.ShapeDtypeStruct((NC * npad,), jnp.float32),
        mesh=_sc_mesh(),
        scratch_types=[
            pltpu.VMEM((nch_deg, K), jnp.int32),
            pltpu.VMEM((128,), jnp.float32),
            pltpu.VMEM((stripe,), jnp.float32),
            pltpu.VMEM_SHARED((npad,), jnp.float32),
        ],
    )
    degp = deg_call(cols_deg, ones128, zeros1)
    degp_t = degp.reshape(NC, npad).T

    nb = 10
    r = n // nb
    h2, xws = pl.pallas_call(
        _dense_body,
        grid=(nb,),
        in_specs=[
            pl.BlockSpec((r, d), lambda i: (i, 0)),
            pl.BlockSpec((d, 2 * d), lambda i: (0, 0)),
            pl.BlockSpec((r, 2), lambda i: (i, 0)),
        ],
        out_specs=[
            pl.BlockSpec((r, d), lambda i: (i, 0)),
            pl.BlockSpec((2, r, half), lambda i: (0, i, 0)),
        ],
        out_shape=[
            jax.ShapeDtypeStruct((n, d), jnp.float32),
            jax.ShapeDtypeStruct((2, n, half), jnp.float32),
        ],
    )(x, wcat, degp_t)

    gcn_call = pl.kernel(
        functools.partial(_gcn_body, npad, nch, ZR),
        out_type=jax.ShapeDtypeStruct((NC, npad, half), jnp.float32),
        mesh=_sc_mesh(),
        scratch_types=[
            [pltpu.VMEM((2, K), jnp.int32) for _ in range(NI)],
            [pltpu.VMEM((K, half), jnp.float32) for _ in range(NG)],
            pltpu.VMEM_SHARED((npad, half), jnp.float32),
            [pltpu.SemaphoreType.DMA for _ in range(NI)],
            [pltpu.SemaphoreType.DMA for _ in range(NG)],
            [pltpu.SemaphoreType.DMA for _ in range(NG)],
        ],
    )
    gcn = gcn_call(xws.reshape(2 * n, half), idx_pack, zeros2)

    out = pl.pallas_call(
        _combine_body,
        grid=(nb,),
        in_specs=[
            pl.BlockSpec((r, d), lambda i: (i, 0)),
            pl.BlockSpec((r, d), lambda i: (i, 0)),
            pl.BlockSpec((2, r, half), lambda i: (0, i, 0)),
            pl.BlockSpec((r, 2), lambda i: (i, 0)),
            pl.BlockSpec((1, d), lambda i: (0, 0)),
        ],
        out_specs=pl.BlockSpec((r, d), lambda i: (i, 0)),
        out_shape=jax.ShapeDtypeStruct((n, d), jnp.float32),
    )(x, h2, gcn, degp_t, bias.reshape(1, d))
    return out
